# baseline (speedup 1.0000x reference)
import functools

import jax
import jax.numpy as jnp
from jax import lax
from jax.experimental import pallas as pl
from jax.experimental.pallas import tpu as pltpu
from jax.experimental.pallas import tpu_sc as plsc

N_ROWS = 20000
N_CLS = 80
N_BOX = 4
CONF = 0.25
NC = 2
NS = 16
NW = NC * NS
ROWS_PER_W = N_ROWS // NW
WIN = 640
L = 16
BIG = 1 << 30

_mesh = plsc.VectorSubcoreMesh(core_axis_name="c", subcore_axis_name="s")


@functools.partial(
    pl.kernel,
    mesh=_mesh,
    out_type=[
        jax.ShapeDtypeStruct((NC, L), jnp.float32),
        jax.ShapeDtypeStruct((NC, L), jnp.int32),
        jax.ShapeDtypeStruct((NC, NS, L), jnp.float32),
        jax.ShapeDtypeStruct((NC, NS, L), jnp.int32),
    ],
    scratch_types=[
        pltpu.VMEM((WIN, N_CLS), jnp.float32),
        pltpu.VMEM((WIN * N_BOX,), jnp.float32),
        pltpu.VMEM((L,), jnp.float32),
        pltpu.VMEM((L,), jnp.int32),
        pltpu.VMEM((NS, L), jnp.float32),
        pltpu.VMEM((NS, L), jnp.int32),
    ],
    compiler_params=pltpu.CompilerParams(needs_layout_passes=False),
)
def _sc_prefix_sum(logits_hbm, boxes_hbm, out_s, out_f, stg_s, stg_f,
                   lg_v, bx_v, vec_f, vec_i, loc_s, loc_f):
    c = lax.axis_index("c")
    s = lax.axis_index("s")
    wid = c * NS + s
    rs = wid * ROWS_PER_W
    ws = (rs // 16) * 16

    pltpu.sync_copy(logits_hbm.at[pl.ds(ws, WIN)], lg_v)
    pltpu.sync_copy(boxes_hbm.at[pl.ds(ws * N_BOX, WIN * N_BOX)], bx_v)

    def body(r, carry):
        still, acc, fail = carry
        g = ws + r
        m = lg_v[r, pl.ds(0, L)]
        for j in range(1, N_CLS // L):
            m = jnp.maximum(m, lg_v[r, pl.ds(j * L, L)])
        score = jnp.max(m)
        in_r = (g >= rs) & (g < rs + ROWS_PER_W)
        ok = score >= CONF
        new_still = still & (ok | jnp.logical_not(in_r))
        take = new_still & in_r
        acc = acc + jnp.where(take, score, jnp.float32(0.0))
        bad = in_r & jnp.logical_not(ok)
        fail = jnp.minimum(fail, jnp.where(bad, g, BIG))
        return new_still, acc, fail

    _, acc, fail = lax.fori_loop(
        0, WIN, body,
        (jnp.bool_(True), jnp.float32(0.0), jnp.int32(BIG)))

    lanes4 = lax.iota(jnp.int32, L) // N_BOX
    limit = jnp.minimum(fail, rs + ROWS_PER_W)

    def bbody(k, bacc):
        v = bx_v[pl.ds(k * L, L)]
        rowv = (ws + k * (L // N_BOX)) + lanes4
        mask = (rowv >= rs) & (rowv < limit)
        return bacc + jnp.where(mask, v, jnp.float32(0.0))

    bacc = lax.fori_loop(0, WIN * N_BOX // L, bbody,
                         jnp.zeros((L,), jnp.float32))
    acc = acc + jnp.sum(bacc)

    vec_f[...] = jnp.full((L,), acc, jnp.float32)
    vec_i[...] = jnp.full((L,), fail, jnp.int32)
    pltpu.sync_copy(vec_f, stg_s.at[c, s])
    pltpu.sync_copy(vec_i, stg_f.at[c, s])
    plsc.subcore_barrier()

    @pl.when(s == 0)
    def _():
        pltpu.sync_copy(stg_s.at[c], loc_s)
        pltpu.sync_copy(stg_f.at[c], loc_f)
        lanes = lax.iota(jnp.int32, L)
        s_coll = jnp.zeros((L,), jnp.float32)
        f_coll = jnp.full((L,), BIG, jnp.int32)
        for u in range(NS):
            s_coll = jnp.where(lanes == u, loc_s[u, pl.ds(0, L)], s_coll)
            f_coll = jnp.where(lanes == u, loc_f[u, pl.ds(0, L)], f_coll)
        core_fail = jnp.min(f_coll)
        starts = (c * NS + lanes) * ROWS_PER_W
        core_sum = jnp.sum(jnp.where(starts <= core_fail, s_coll,
                                     jnp.float32(0.0)))
        vec_f[...] = jnp.full((L,), core_sum, jnp.float32)
        vec_i[...] = jnp.full((L,), core_fail, jnp.int32)
        pltpu.sync_copy(vec_f, out_s.at[c])
        pltpu.sync_copy(vec_i, out_f.at[c])


def kernel(logits, boxes):
    sums, fails, _, _ = _sc_prefix_sum(logits, boxes.reshape(-1))
    s0 = sums[0, 0]
    s1 = sums[1, 0]
    f0 = fails[0, 0]
    total = s0 + jnp.where(f0 >= NS * ROWS_PER_W, s1, jnp.float32(0.0))
    return total.astype(jnp.float32)
